# baseline (speedup 1.0000x reference)
import functools

import jax
import jax.numpy as jnp
from jax import lax
from jax.experimental import pallas as pl
from jax.experimental.pallas import tpu as pltpu
from jax.experimental.pallas import tpu_sc as plsc

N = 10000
E = 320000
NC = 2
NS = 16
NW = NC * NS
BN = 1000
ROWS_CHUNK = 200
NROW_CHUNKS = N // ROWS_CHUNK
EDGE_B = 128


def _make_segsum(d):
    ept = E // NW
    nchunk = ept // EDGE_B
    tail = ept - nchunk * EDGE_B
    nbuf = 4 if d > 48 else 6
    la = nbuf // 2
    ngroups = (nchunk + nbuf - 1) // nbuf
    mesh = plsc.VectorSubcoreMesh(core_axis_name="c", subcore_axis_name="s")

    @functools.partial(
        pl.kernel,
        mesh=mesh,
        compiler_params=pltpu.CompilerParams(use_tc_tiling_on_sc=False),
        out_type=jax.ShapeDtypeStruct((NC, N, d), jnp.float32),
        scratch_types=[
            pltpu.VMEM((ept,), jnp.int32),
            pltpu.VMEM((ept,), jnp.int32),
            pltpu.VMEM((nbuf, EDGE_B, d), jnp.float32),
            pltpu.VMEM((max(tail, 1), d), jnp.float32),
            pltpu.VMEM((ROWS_CHUNK, d), jnp.float32),
            pltpu.VMEM_SHARED((N, d), jnp.float32),
            pltpu.SemaphoreType.DMA((nbuf,)),
            pltpu.SemaphoreType.DMA((nbuf,)),
            pltpu.SemaphoreType.DMA,
        ],
    )
    def seg(tab, ei, zero, out,
            src_v, dst_v, rows, trows, bounce_v, accum, gsem, ssem, tsem):
        c = lax.axis_index("c")
        s = lax.axis_index("s")
        t = c * NS + s
        e0 = pl.multiple_of(t * ept, EDGE_B)
        pltpu.sync_copy(ei.at[0, pl.ds(e0, ept)], src_v)
        pltpu.sync_copy(ei.at[1, pl.ds(e0, ept)], dst_v)
        pltpu.sync_copy(zero, bounce_v)
        for k in range(4):
            chunk = s + k * NS

            @pl.when(chunk < NROW_CHUNKS)
            def _():
                r = pl.multiple_of(chunk * ROWS_CHUNK, ROWS_CHUNK)
                pltpu.sync_copy(bounce_v, accum.at[pl.ds(r, ROWS_CHUNK)])

        plsc.subcore_barrier()

        def gfire(j, p):
            pltpu.async_copy(tab.at[src_v.at[pl.ds(j * EDGE_B, EDGE_B)]], rows.at[p], gsem.at[p])

        def gwait(j, p):
            pltpu.make_async_copy(
                tab.at[src_v.at[pl.ds(j * EDGE_B, EDGE_B)]], rows.at[p], gsem.at[p]).wait()

        def sfire(j, p):
            pltpu.async_copy(rows.at[p], accum.at[dst_v.at[pl.ds(j * EDGE_B, EDGE_B)]], ssem.at[p],
                             add=True)

        def swait(j, p):
            pltpu.make_async_copy(
                rows.at[p], accum.at[dst_v.at[pl.ds(j * EDGE_B, EDGE_B)]], ssem.at[p]).wait()

        for p in range(la):
            gfire(p, p)

        def body(k, carry):
            for p in range(nbuf):
                j = nbuf * k + p
                pf = (p + la) % nbuf

                @pl.when(jnp.logical_and(j >= la, j < nchunk + la))
                def _():
                    swait(j - la, pf)

                @pl.when(j + la < nchunk)
                def _():
                    gfire(j + la, pf)

                @pl.when(j < nchunk)
                def _():
                    gwait(j, p)
                    sfire(j, p)

            return carry

        if tail:
            tb = nchunk * EDGE_B
            pltpu.async_copy(
                tab.at[src_v.at[pl.ds(tb, tail)]], trows, tsem)

        lax.fori_loop(0, ngroups, body, 0)
        for j in range(max(nbuf * ngroups - la, 0), nchunk):
            swait(j, j % nbuf)
        if tail:
            tb = nchunk * EDGE_B
            pltpu.make_async_copy(
                tab.at[src_v.at[pl.ds(tb, tail)]], trows, tsem).wait()
            pltpu.async_copy(
                trows, accum.at[dst_v.at[pl.ds(tb, tail)]], tsem, add=True)
            pltpu.make_async_copy(
                trows, accum.at[dst_v.at[pl.ds(tb, tail)]], tsem).wait()
        plsc.subcore_barrier()

        for k in range(4):
            chunk = s + k * NS

            @pl.when(chunk < NROW_CHUNKS)
            def _():
                r = pl.multiple_of(chunk * ROWS_CHUNK, ROWS_CHUNK)
                pltpu.sync_copy(accum.at[pl.ds(r, ROWS_CHUNK)], bounce_v)
                pltpu.sync_copy(bounce_v, out.at[c, pl.ds(r, ROWS_CHUNK)])

    return seg


_segsum80 = _make_segsum(80)
_segsum32 = _make_segsum(32)


def _t0_body(x_ref, wn_ref, ws_ref, zp_ref, s_ref):
    xb = x_ref[...]
    z = jnp.dot(xb, wn_ref[...], preferred_element_type=jnp.float32)
    zp_ref[...] = jnp.concatenate(
        [z, jnp.ones((BN, 16), jnp.float32)], axis=1)
    s_ref[...] = jnp.dot(xb, ws_ref[...], preferred_element_type=jnp.float32)


def _c1a_body(a_ref, s0_ref, b0_ref, wn_ref, z1_ref, h1_ref, inv_ref):
    a = a_ref[0] + a_ref[1]
    inv = 1.0 / jnp.maximum(a[:, 64:65], 1.0)
    h1 = jnp.maximum(s0_ref[...] + a[:, :64] * inv + b0_ref[...], 0.0)
    z1_ref[...] = jnp.dot(h1, wn_ref[...], preferred_element_type=jnp.float32)
    h1_ref[...] = h1
    inv_ref[...] = jnp.broadcast_to(inv, (BN, 8))


def _c1b_body(h1_ref, ws_ref, s1_ref):
    s1_ref[...] = jnp.dot(h1_ref[...], ws_ref[...],
                          preferred_element_type=jnp.float32)


def _c2a_body(a_ref, s1_ref, b1_ref, inv_ref, h2_ref):
    a = a_ref[0] + a_ref[1]
    h2_ref[...] = jnp.maximum(
        s1_ref[...] + a * inv_ref[:, 0:1] + b1_ref[...], 0.0)


def _c2b_body(h2_ref, ws_ref, s2_ref):
    s2_ref[...] = jnp.dot(h2_ref[...], ws_ref[...],
                          preferred_element_type=jnp.float32)


def _c3_body(a_ref, s2_ref, inv_ref, wn_ref, b2_ref, o_ref):
    hn = (a_ref[0] + a_ref[1]) * inv_ref[:, 0:1]
    o_ref[...] = jnp.maximum(
        s2_ref[...]
        + jnp.dot(hn, wn_ref[...], preferred_element_type=jnp.float32)
        + b2_ref[...], 0.0)


def _row_spec(w):
    return pl.BlockSpec((BN, w), lambda i: (i, 0))


def _full_spec(shape):
    nd = len(shape)
    return pl.BlockSpec(shape, lambda i: (0,) * nd)


def _part_spec(w):
    return pl.BlockSpec((NC, BN, w), lambda i: (0, i, 0))


_GRID = (N // BN,)


def _tc(body, in_specs, out_specs, out_shape):
    return pl.pallas_call(body, grid=_GRID, in_specs=in_specs,
                          out_specs=out_specs, out_shape=out_shape)


def kernel(x, edge_index, W_self0, W_neigh0, b0,
           W_self1, W_neigh1, b1, W_self2, W_neigh2, b2):
    zero80 = jnp.zeros((ROWS_CHUNK, 80), jnp.float32)
    zero32 = jnp.zeros((ROWS_CHUNK, 32), jnp.float32)

    z0p, s0 = _tc(
        _t0_body,
        [_row_spec(128), _full_spec((128, 64)), _full_spec((128, 64))],
        [_row_spec(80), _row_spec(64)],
        [jax.ShapeDtypeStruct((N, 80), jnp.float32),
         jax.ShapeDtypeStruct((N, 64), jnp.float32)],
    )(x, W_neigh0, W_self0)

    a0 = _segsum80(z0p, edge_index, zero80)

    z1, h1, invd = _tc(
        _c1a_body,
        [_part_spec(80), _row_spec(64), _full_spec((1, 64)),
         _full_spec((64, 32))],
        [_row_spec(32), _row_spec(64), _row_spec(8)],
        [jax.ShapeDtypeStruct((N, 32), jnp.float32),
         jax.ShapeDtypeStruct((N, 64), jnp.float32),
         jax.ShapeDtypeStruct((N, 8), jnp.float32)],
    )(a0, s0, b0.reshape(1, 64), W_neigh1)

    a1 = _segsum32(z1, edge_index, zero32)

    (s1,) = _tc(
        _c1b_body,
        [_row_spec(64), _full_spec((64, 32))],
        [_row_spec(32)],
        [jax.ShapeDtypeStruct((N, 32), jnp.float32)],
    )(h1, W_self1)

    (h2,) = _tc(
        _c2a_body,
        [_part_spec(32), _row_spec(32), _full_spec((1, 32)), _row_spec(8)],
        [_row_spec(32)],
        [jax.ShapeDtypeStruct((N, 32), jnp.float32)],
    )(a1, s1, b1.reshape(1, 32), invd)

    a2 = _segsum32(h2, edge_index, zero32)

    (s2,) = _tc(
        _c2b_body,
        [_row_spec(32), _full_spec((32, 128))],
        [_row_spec(128)],
        [jax.ShapeDtypeStruct((N, 128), jnp.float32)],
    )(h2, W_self2)

    (out,) = _tc(
        _c3_body,
        [_part_spec(32), _row_spec(128), _row_spec(8),
         _full_spec((32, 128)), _full_spec((1, 128))],
        [_row_spec(128)],
        [jax.ShapeDtypeStruct((N, 128), jnp.float32)],
    )(a2, s2, invd, W_neigh2, b2.reshape(1, 128))

    return out


# speedup vs baseline: 1.0250x; 1.0250x over previous
import functools

import jax
import jax.numpy as jnp
from jax import lax
from jax.experimental import pallas as pl
from jax.experimental.pallas import tpu as pltpu
from jax.experimental.pallas import tpu_sc as plsc

N = 10000
E = 320000
NC = 2
NS = 16
NW = NC * NS
BN = 1000
ROWS_CHUNK = 200
NROW_CHUNKS = N // ROWS_CHUNK
EDGE_B = 128


def _make_segsum(d, edge_b, nbuf):
    EDGE_B = edge_b
    ept = E // NW
    nchunk = ept // EDGE_B
    tail = ept - nchunk * EDGE_B
    la = nbuf // 2
    ngroups = (nchunk + nbuf - 1) // nbuf
    mesh = plsc.VectorSubcoreMesh(core_axis_name="c", subcore_axis_name="s")

    @functools.partial(
        pl.kernel,
        mesh=mesh,
        compiler_params=pltpu.CompilerParams(use_tc_tiling_on_sc=False),
        out_type=jax.ShapeDtypeStruct((NC, N, d), jnp.float32),
        scratch_types=[
            pltpu.VMEM((ept,), jnp.int32),
            pltpu.VMEM((ept,), jnp.int32),
            pltpu.VMEM((nbuf, EDGE_B, d), jnp.float32),
            pltpu.VMEM((max(tail, 1), d), jnp.float32),
            pltpu.VMEM((ROWS_CHUNK, d), jnp.float32),
            pltpu.VMEM_SHARED((N, d), jnp.float32),
            pltpu.SemaphoreType.DMA((nbuf,)),
            pltpu.SemaphoreType.DMA((nbuf,)),
            pltpu.SemaphoreType.DMA,
        ],
    )
    def seg(tab, ei, zero, out,
            src_v, dst_v, rows, trows, bounce_v, accum, gsem, ssem, tsem):
        c = lax.axis_index("c")
        s = lax.axis_index("s")
        t = c * NS + s
        e0 = pl.multiple_of(t * ept, EDGE_B)
        pltpu.sync_copy(ei.at[0, pl.ds(e0, ept)], src_v)
        pltpu.sync_copy(ei.at[1, pl.ds(e0, ept)], dst_v)
        pltpu.sync_copy(zero, bounce_v)
        for k in range(4):
            chunk = s + k * NS

            @pl.when(chunk < NROW_CHUNKS)
            def _():
                r = pl.multiple_of(chunk * ROWS_CHUNK, ROWS_CHUNK)
                pltpu.sync_copy(bounce_v, accum.at[pl.ds(r, ROWS_CHUNK)])

        plsc.subcore_barrier()

        def gfire(j, p):
            pltpu.async_copy(tab.at[src_v.at[pl.ds(j * EDGE_B, EDGE_B)]], rows.at[p], gsem.at[p])

        def gwait(j, p):
            pltpu.make_async_copy(
                tab.at[src_v.at[pl.ds(j * EDGE_B, EDGE_B)]], rows.at[p], gsem.at[p]).wait()

        def sfire(j, p):
            pltpu.async_copy(rows.at[p], accum.at[dst_v.at[pl.ds(j * EDGE_B, EDGE_B)]], ssem.at[p],
                             add=True)

        def swait(j, p):
            pltpu.make_async_copy(
                rows.at[p], accum.at[dst_v.at[pl.ds(j * EDGE_B, EDGE_B)]], ssem.at[p]).wait()

        for p in range(la):
            gfire(p, p)

        def body(k, carry):
            for p in range(nbuf):
                j = nbuf * k + p
                pf = (p + la) % nbuf

                @pl.when(jnp.logical_and(j >= la, j < nchunk + la))
                def _():
                    swait(j - la, pf)

                @pl.when(j + la < nchunk)
                def _():
                    gfire(j + la, pf)

                @pl.when(j < nchunk)
                def _():
                    gwait(j, p)
                    sfire(j, p)

            return carry

        if tail:
            tb = nchunk * EDGE_B
            pltpu.async_copy(
                tab.at[src_v.at[pl.ds(tb, tail)]], trows, tsem)

        lax.fori_loop(0, ngroups, body, 0)
        for j in range(max(nbuf * ngroups - la, 0), nchunk):
            swait(j, j % nbuf)
        if tail:
            tb = nchunk * EDGE_B
            pltpu.make_async_copy(
                tab.at[src_v.at[pl.ds(tb, tail)]], trows, tsem).wait()
            pltpu.async_copy(
                trows, accum.at[dst_v.at[pl.ds(tb, tail)]], tsem, add=True)
            pltpu.make_async_copy(
                trows, accum.at[dst_v.at[pl.ds(tb, tail)]], tsem).wait()
        plsc.subcore_barrier()

        for k in range(4):
            chunk = s + k * NS

            @pl.when(chunk < NROW_CHUNKS)
            def _():
                r = pl.multiple_of(chunk * ROWS_CHUNK, ROWS_CHUNK)
                pltpu.sync_copy(accum.at[pl.ds(r, ROWS_CHUNK)], bounce_v)
                pltpu.sync_copy(bounce_v, out.at[c, pl.ds(r, ROWS_CHUNK)])

    return seg


_segsum80 = _make_segsum(80, 80, 6)
_segsum32 = _make_segsum(32, 128, 6)


def _t0_body(x_ref, wn_ref, ws_ref, zp_ref, s_ref):
    xb = x_ref[...]
    z = jnp.dot(xb, wn_ref[...], preferred_element_type=jnp.float32)
    zp_ref[...] = jnp.concatenate(
        [z, jnp.ones((BN, 16), jnp.float32)], axis=1)
    s_ref[...] = jnp.dot(xb, ws_ref[...], preferred_element_type=jnp.float32)


def _c1a_body(a_ref, s0_ref, b0_ref, wn_ref, z1_ref, h1_ref, inv_ref):
    a = a_ref[0] + a_ref[1]
    inv = 1.0 / jnp.maximum(a[:, 64:65], 1.0)
    h1 = jnp.maximum(s0_ref[...] + a[:, :64] * inv + b0_ref[...], 0.0)
    z1_ref[...] = jnp.dot(h1, wn_ref[...], preferred_element_type=jnp.float32)
    h1_ref[...] = h1
    inv_ref[...] = jnp.broadcast_to(inv, (BN, 8))


def _c1b_body(h1_ref, ws_ref, s1_ref):
    s1_ref[...] = jnp.dot(h1_ref[...], ws_ref[...],
                          preferred_element_type=jnp.float32)


def _c2a_body(a_ref, s1_ref, b1_ref, inv_ref, h2_ref):
    a = a_ref[0] + a_ref[1]
    h2_ref[...] = jnp.maximum(
        s1_ref[...] + a * inv_ref[:, 0:1] + b1_ref[...], 0.0)


def _c2b_body(h2_ref, ws_ref, s2_ref):
    s2_ref[...] = jnp.dot(h2_ref[...], ws_ref[...],
                          preferred_element_type=jnp.float32)


def _c3_body(a_ref, s2_ref, inv_ref, wn_ref, b2_ref, o_ref):
    hn = (a_ref[0] + a_ref[1]) * inv_ref[:, 0:1]
    o_ref[...] = jnp.maximum(
        s2_ref[...]
        + jnp.dot(hn, wn_ref[...], preferred_element_type=jnp.float32)
        + b2_ref[...], 0.0)


def _row_spec(w):
    return pl.BlockSpec((BN, w), lambda i: (i, 0))


def _full_spec(shape):
    nd = len(shape)
    return pl.BlockSpec(shape, lambda i: (0,) * nd)


def _part_spec(w):
    return pl.BlockSpec((NC, BN, w), lambda i: (0, i, 0))


_GRID = (N // BN,)


def _tc(body, in_specs, out_specs, out_shape):
    return pl.pallas_call(body, grid=_GRID, in_specs=in_specs,
                          out_specs=out_specs, out_shape=out_shape)


def kernel(x, edge_index, W_self0, W_neigh0, b0,
           W_self1, W_neigh1, b1, W_self2, W_neigh2, b2):
    zero80 = jnp.zeros((ROWS_CHUNK, 80), jnp.float32)
    zero32 = jnp.zeros((ROWS_CHUNK, 32), jnp.float32)

    z0p, s0 = _tc(
        _t0_body,
        [_row_spec(128), _full_spec((128, 64)), _full_spec((128, 64))],
        [_row_spec(80), _row_spec(64)],
        [jax.ShapeDtypeStruct((N, 80), jnp.float32),
         jax.ShapeDtypeStruct((N, 64), jnp.float32)],
    )(x, W_neigh0, W_self0)

    a0 = _segsum80(z0p, edge_index, zero80)

    z1, h1, invd = _tc(
        _c1a_body,
        [_part_spec(80), _row_spec(64), _full_spec((1, 64)),
         _full_spec((64, 32))],
        [_row_spec(32), _row_spec(64), _row_spec(8)],
        [jax.ShapeDtypeStruct((N, 32), jnp.float32),
         jax.ShapeDtypeStruct((N, 64), jnp.float32),
         jax.ShapeDtypeStruct((N, 8), jnp.float32)],
    )(a0, s0, b0.reshape(1, 64), W_neigh1)

    a1 = _segsum32(z1, edge_index, zero32)

    (s1,) = _tc(
        _c1b_body,
        [_row_spec(64), _full_spec((64, 32))],
        [_row_spec(32)],
        [jax.ShapeDtypeStruct((N, 32), jnp.float32)],
    )(h1, W_self1)

    (h2,) = _tc(
        _c2a_body,
        [_part_spec(32), _row_spec(32), _full_spec((1, 32)), _row_spec(8)],
        [_row_spec(32)],
        [jax.ShapeDtypeStruct((N, 32), jnp.float32)],
    )(a1, s1, b1.reshape(1, 32), invd)

    a2 = _segsum32(h2, edge_index, zero32)

    (s2,) = _tc(
        _c2b_body,
        [_row_spec(32), _full_spec((32, 128))],
        [_row_spec(128)],
        [jax.ShapeDtypeStruct((N, 128), jnp.float32)],
    )(h2, W_self2)

    (out,) = _tc(
        _c3_body,
        [_part_spec(32), _row_spec(128), _row_spec(8),
         _full_spec((32, 128)), _full_spec((1, 128))],
        [_row_spec(128)],
        [jax.ShapeDtypeStruct((N, 128), jnp.float32)],
    )(a2, s2, invd, W_neigh2, b2.reshape(1, 128))

    return out
